# baseline (speedup 1.0000x reference)
import functools

import jax
import jax.numpy as jnp
from jax import lax
from jax.experimental import pallas as pl
from jax.experimental.pallas import tpu as pltpu
from jax.experimental.pallas import tpu_sc as plsc

_NUM_CORES = 2
_NUM_SUBCORES = 16
_NUM_WORKERS = _NUM_CORES * _NUM_SUBCORES
_CHUNK = 128
_K = 2
_LANES = 16


@functools.lru_cache(maxsize=None)
def _make_phase_a(V, D):
    n_slabs = -(-V // _CHUNK)
    n_iter = -(-n_slabs // _NUM_WORKERS)
    g_rows = V // 2
    mesh = plsc.VectorSubcoreMesh(core_axis_name="c", subcore_axis_name="s")

    @functools.partial(
        pl.kernel,
        mesh=mesh,
        out_type=jax.ShapeDtypeStruct((g_rows, 2 * D), jnp.float32),
        scratch_types=[
            pltpu.VMEM((2, D, _CHUNK), jnp.float32),
            pltpu.VMEM((2, _CHUNK // 2, 2 * D), jnp.float32),
            pltpu.SemaphoreType.DMA,
            pltpu.SemaphoreType.DMA,
        ],
        compiler_params=pltpu.CompilerParams(
            use_tc_tiling_on_sc=True, needs_layout_passes=False),
    )
    def phase_a(wt_hbm, tail_hbm, g_hbm, slab_v, pack_v, ls, ws):
        wid = lax.axis_index("s") * _NUM_CORES + lax.axis_index("c")

        lane = lax.iota(jnp.int32, _LANES)

        def c_of(t):
            return wid + t * _NUM_WORKERS

        def fire_load(t, buf):
            c = c_of(t)

            @pl.when(c < n_slabs - 1)
            def _():
                col = pl.multiple_of(c * _CHUNK, _CHUNK)
                pltpu.async_copy(wt_hbm.at[:, pl.ds(col, _CHUNK)],
                                 slab_v.at[buf], ls)

            @pl.when(c == n_slabs - 1)
            def _():
                pltpu.async_copy(tail_hbm, slab_v.at[buf], ls)

        def wait_load(t, buf):
            pltpu.make_async_copy(tail_hbm, slab_v.at[buf], ls).wait()

        def row0_of(t):
            c = c_of(t)
            return pl.multiple_of(
                lax.min(c * (_CHUNK // 2), (V - _CHUNK) // 2), 8)

        def fire_store(t, buf):
            pltpu.async_copy(pack_v.at[buf],
                             g_hbm.at[pl.ds(row0_of(t), _CHUNK // 2)], ws)

        def wait_store(t, buf):
            pltpu.make_async_copy(
                pack_v.at[buf],
                g_hbm.at[pl.ds(row0_of(t), _CHUNK // 2)], ws).wait()

        def transpose_slab(buf):
            def tbody(blk, carry):
                dv = lane + blk * _LANES
                off = blk * _LANES
                for j in range(_CHUNK):
                    jv = jnp.zeros((_LANES,), jnp.int32) + j
                    v = plsc.load_gather(slab_v.at[buf], [dv, jv])
                    pack_v[buf, j // 2,
                           pl.ds((j % 2) * D + off, _LANES)] = v
                return carry

            lax.fori_loop(0, D // _LANES, tbody, 0)

        c0_valid = wid < n_slabs

        @pl.when(c0_valid)
        def _():
            fire_load(0, 0)

        def step(t, carry):
            c = c_of(t)
            valid = c < n_slabs
            nxt_valid = c + _NUM_WORKERS < n_slabs

            @pl.when(valid)
            def _():
                for b in range(2):
                    @pl.when(lax.rem(t, 2) == b)
                    def _():
                        wait_load(t, b)

                        @pl.when(nxt_valid)
                        def _():
                            fire_load(t + 1, 1 - b)

                        @pl.when(t >= 2)
                        def _():
                            wait_store(t - 2, b)

                        transpose_slab(b)
                        fire_store(t, b)
            return carry

        lax.fori_loop(0, n_iter, step, 0)

        for t in range(max(0, n_iter - 3), n_iter):
            c = c_of(t)
            c2 = c_of(t + 2)

            @pl.when((c < n_slabs) & (c2 >= n_slabs))
            def _():
                for b in range(2):
                    @pl.when(lax.rem(t, 2) == b)
                    def _():
                        wait_store(t, b)

    return phase_a


@functools.lru_cache(maxsize=None)
def _make_phase_b(V, D, S, B0, g_rows):
    b_per_w = B0 // _NUM_WORKERS
    k_per_s = b_per_w // _CHUNK
    n_tiles = S * k_per_s
    n_groups = n_tiles // _K
    assert n_groups >= 4
    mesh = plsc.VectorSubcoreMesh(core_axis_name="c", subcore_axis_name="s")

    @functools.partial(
        pl.kernel,
        mesh=mesh,
        out_type=jax.ShapeDtypeStruct((S, D, B0), jnp.float32),
        scratch_types=[
            pltpu.VMEM((n_tiles, _CHUNK), jnp.int32),
            pltpu.VMEM((2, _K, _CHUNK), jnp.int32),
            pltpu.VMEM((2, _K, _CHUNK, 2 * D), jnp.float32),
            pltpu.VMEM((2, _K, D, _CHUNK), jnp.float32),
            pltpu.SemaphoreType.DMA,
            pltpu.SemaphoreType.DMA,
            pltpu.SemaphoreType.DMA,
            pltpu.SemaphoreType.DMA,
        ],
        compiler_params=pltpu.CompilerParams(
            use_tc_tiling_on_sc=True, needs_layout_passes=False),
    )
    def phase_b(idx_hbm, g_hbm, out_hbm, idx_v, pidx_v, rows_v, tsp_v,
                gs0, gs1, os0, os1):
        wid = lax.axis_index("s") * _NUM_CORES + lax.axis_index("c")
        col0 = wid * b_per_w
        gsems = (gs0, gs1)
        osems = (os0, os1)
        pltpu.sync_copy(idx_hbm.at[wid], idx_v)

        lane = lax.iota(jnp.int32, _LANES)
        rvec = [lane + m * _LANES for m in range(_CHUNK // _LANES)]

        def tile_sb(t):
            return t // k_per_s, (t % k_per_s) * _CHUNK

        def fire_gathers(g, x):
            for b in range(_K):
                t = g * _K + b
                for m in range(_CHUNK // _LANES):
                    ids = idx_v[t, pl.ds(m * _LANES, _LANES)]
                    pidx_v[x, b, pl.ds(m * _LANES, _LANES)] = (
                        lax.shift_right_logical(ids, 1))
                pltpu.async_copy(g_hbm.at[pidx_v.at[x, b]],
                                 rows_v.at[x, b], gsems[x])

        def drain_gathers(g, x):
            for b in range(_K):
                pltpu.make_async_copy(g_hbm.at[pidx_v.at[x, b]],
                                      rows_v.at[x, b], gsems[x]).wait()

        def transpose_group(g, x):
            for b in range(_K):
                t = g * _K + b
                half = []
                for m in range(_CHUNK // _LANES):
                    ids = idx_v[t, pl.ds(m * _LANES, _LANES)]
                    half.append((ids & 1) * D)

                def tbody(blk, carry):
                    hb = [h + blk * _LANES for h in half]
                    for dd in range(_LANES):
                        d = blk * _LANES + dd
                        for m in range(_CHUNK // _LANES):
                            v = plsc.load_gather(rows_v.at[x, b],
                                                 [rvec[m], hb[m] + dd])
                            tsp_v[x, b, d, pl.ds(m * _LANES, _LANES)] = v
                    return carry

                lax.fori_loop(0, D // _LANES, tbody, 0)

        def fire_writes(g, x):
            for b in range(_K):
                s, boff = tile_sb(g * _K + b)
                col = pl.multiple_of(col0 + boff, _CHUNK)
                pltpu.async_copy(tsp_v.at[x, b],
                                 out_hbm.at[s, :, pl.ds(col, _CHUNK)],
                                 osems[x])

        def drain_writes(g, x):
            for b in range(_K):
                s, boff = tile_sb(g * _K + b)
                col = pl.multiple_of(col0 + boff, _CHUNK)
                pltpu.make_async_copy(
                    tsp_v.at[x, b],
                    out_hbm.at[s, :, pl.ds(col, _CHUNK)], osems[x]).wait()

        fire_gathers(0, 0)
        fire_gathers(1, 1)

        def body(g, carry):
            for x in range(2):
                @pl.when(lax.rem(g, 2) == x)
                def _():
                    drain_gathers(g, x)

                    @pl.when(g >= 2)
                    def _():
                        drain_writes(g - 2, x)

                    transpose_group(g, x)
                    fire_writes(g, x)

                    @pl.when(g + 2 < n_groups)
                    def _():
                        fire_gathers(g + 2, x)
            return carry

        lax.fori_loop(0, n_groups, body, 0)

        drain_writes(n_groups - 2, (n_groups - 2) % 2)
        drain_writes(n_groups - 1, (n_groups - 1) % 2)

    return phase_b


def kernel(token_ids, weight):
    B0, S = token_ids.shape
    V, D = weight.shape
    wt = weight.T
    tail = lax.slice(wt, (0, V - _CHUNK), (D, V))
    g = _make_phase_a(V, D)(wt, tail)
    g_rows = g.shape[0]

    tids = token_ids.T.astype(jnp.int32)
    b_per_w = B0 // _NUM_WORKERS
    idx3 = (tids.reshape(S, _NUM_WORKERS, b_per_w // _CHUNK, _CHUNK)
            .transpose(1, 0, 2, 3)
            .reshape(_NUM_WORKERS, S * (b_per_w // _CHUNK), _CHUNK))
    out_t = _make_phase_b(V, D, S, B0, g_rows)(idx3, g)
    return jnp.transpose(out_t, (2, 0, 1))


# speedup vs baseline: 1.4963x; 1.4963x over previous
import functools

import jax
import jax.numpy as jnp
from jax import lax
from jax.experimental import pallas as pl
from jax.experimental.pallas import tpu as pltpu
from jax.experimental.pallas import tpu_sc as plsc

_NUM_CORES = 2
_NUM_SUBCORES = 16
_NUM_WORKERS = _NUM_CORES * _NUM_SUBCORES
_CHUNK = 128
_K = 2
_LANES = 16


@functools.lru_cache(maxsize=None)
def _make_phase_a(V, D):
    n_slabs = -(-V // _CHUNK)
    n_iter = -(-n_slabs // _NUM_WORKERS)
    g_rows = V // 2
    mesh = plsc.VectorSubcoreMesh(core_axis_name="c", subcore_axis_name="s")

    @functools.partial(
        pl.kernel,
        mesh=mesh,
        out_type=jax.ShapeDtypeStruct((g_rows, 2 * D), jnp.float32),
        scratch_types=[
            pltpu.VMEM((2, D, _CHUNK), jnp.float32),
            pltpu.VMEM((2, _CHUNK // 2, 2 * D), jnp.float32),
            pltpu.SemaphoreType.DMA,
            pltpu.SemaphoreType.DMA,
        ],
        compiler_params=pltpu.CompilerParams(
            use_tc_tiling_on_sc=True, needs_layout_passes=False),
    )
    def phase_a(wt_hbm, tail_hbm, g_hbm, slab_v, pack_v, ls, ws):
        wid = lax.axis_index("s") * _NUM_CORES + lax.axis_index("c")

        lane = lax.iota(jnp.int32, _LANES)

        def c_of(t):
            return wid + t * _NUM_WORKERS

        def fire_load(t, buf):
            c = c_of(t)

            @pl.when(c < n_slabs - 1)
            def _():
                col = pl.multiple_of(c * _CHUNK, _CHUNK)
                pltpu.async_copy(wt_hbm.at[:, pl.ds(col, _CHUNK)],
                                 slab_v.at[buf], ls)

            @pl.when(c == n_slabs - 1)
            def _():
                pltpu.async_copy(tail_hbm, slab_v.at[buf], ls)

        def wait_load(t, buf):
            pltpu.make_async_copy(tail_hbm, slab_v.at[buf], ls).wait()

        def row0_of(t):
            c = c_of(t)
            return pl.multiple_of(
                lax.min(c * (_CHUNK // 2), (V - _CHUNK) // 2), 8)

        def fire_store(t, buf):
            pltpu.async_copy(pack_v.at[buf],
                             g_hbm.at[pl.ds(row0_of(t), _CHUNK // 2)], ws)

        def wait_store(t, buf):
            pltpu.make_async_copy(
                pack_v.at[buf],
                g_hbm.at[pl.ds(row0_of(t), _CHUNK // 2)], ws).wait()

        def transpose_slab(buf):
            def tbody(blk, carry):
                dv = lane + blk * _LANES
                off = blk * _LANES
                for j0 in range(0, _CHUNK, 8):
                    vs = []
                    for j in range(j0, j0 + 8):
                        jv = jnp.zeros((_LANES,), jnp.int32) + j
                        vs.append(plsc.load_gather(slab_v.at[buf], [dv, jv]))
                    for i, j in enumerate(range(j0, j0 + 8)):
                        pack_v[buf, j // 2,
                               pl.ds((j % 2) * D + off, _LANES)] = vs[i]
                return carry

            lax.fori_loop(0, D // _LANES, tbody, 0)

        c0_valid = wid < n_slabs

        @pl.when(c0_valid)
        def _():
            fire_load(0, 0)

        def step(t, carry):
            c = c_of(t)
            valid = c < n_slabs
            nxt_valid = c + _NUM_WORKERS < n_slabs

            @pl.when(valid)
            def _():
                for b in range(2):
                    @pl.when(lax.rem(t, 2) == b)
                    def _():
                        wait_load(t, b)

                        @pl.when(nxt_valid)
                        def _():
                            fire_load(t + 1, 1 - b)

                        @pl.when(t >= 2)
                        def _():
                            wait_store(t - 2, b)

                        transpose_slab(b)
                        fire_store(t, b)
            return carry

        lax.fori_loop(0, n_iter, step, 0)

        for t in range(max(0, n_iter - 3), n_iter):
            c = c_of(t)
            c2 = c_of(t + 2)

            @pl.when((c < n_slabs) & (c2 >= n_slabs))
            def _():
                for b in range(2):
                    @pl.when(lax.rem(t, 2) == b)
                    def _():
                        wait_store(t, b)

    return phase_a


@functools.lru_cache(maxsize=None)
def _make_phase_b(V, D, S, B0, g_rows):
    b_per_w = B0 // _NUM_WORKERS
    k_per_s = b_per_w // _CHUNK
    n_tiles = S * k_per_s
    n_groups = n_tiles // _K
    assert n_groups >= 4
    mesh = plsc.VectorSubcoreMesh(core_axis_name="c", subcore_axis_name="s")

    @functools.partial(
        pl.kernel,
        mesh=mesh,
        out_type=jax.ShapeDtypeStruct((S, D, B0), jnp.float32),
        scratch_types=[
            pltpu.VMEM((n_tiles, _CHUNK), jnp.int32),
            pltpu.VMEM((2, _K, _CHUNK), jnp.int32),
            pltpu.VMEM((2, _K, _CHUNK, 2 * D), jnp.float32),
            pltpu.VMEM((2, _K, D, _CHUNK), jnp.float32),
            pltpu.SemaphoreType.DMA,
            pltpu.SemaphoreType.DMA,
            pltpu.SemaphoreType.DMA,
            pltpu.SemaphoreType.DMA,
        ],
        compiler_params=pltpu.CompilerParams(
            use_tc_tiling_on_sc=True, needs_layout_passes=False),
    )
    def phase_b(idx_hbm, g_hbm, out_hbm, idx_v, pidx_v, rows_v, tsp_v,
                gs0, gs1, os0, os1):
        wid = lax.axis_index("s") * _NUM_CORES + lax.axis_index("c")
        col0 = wid * b_per_w
        gsems = (gs0, gs1)
        osems = (os0, os1)
        pltpu.sync_copy(idx_hbm.at[wid], idx_v)

        lane = lax.iota(jnp.int32, _LANES)
        rvec = [lane + m * _LANES for m in range(_CHUNK // _LANES)]

        def tile_sb(t):
            return t // k_per_s, (t % k_per_s) * _CHUNK

        def fire_gathers(g, x):
            for b in range(_K):
                t = g * _K + b
                for m in range(_CHUNK // _LANES):
                    ids = idx_v[t, pl.ds(m * _LANES, _LANES)]
                    pidx_v[x, b, pl.ds(m * _LANES, _LANES)] = (
                        lax.shift_right_logical(ids, 1))
                pltpu.async_copy(g_hbm.at[pidx_v.at[x, b]],
                                 rows_v.at[x, b], gsems[x])

        def drain_gathers(g, x):
            for b in range(_K):
                pltpu.make_async_copy(g_hbm.at[pidx_v.at[x, b]],
                                      rows_v.at[x, b], gsems[x]).wait()

        def transpose_group(g, x):
            for b in range(_K):
                t = g * _K + b
                half = []
                for m in range(_CHUNK // _LANES):
                    ids = idx_v[t, pl.ds(m * _LANES, _LANES)]
                    half.append((ids & 1) * D)

                def tbody(blk, carry):
                    hb = [h + blk * _LANES for h in half]
                    for dd in range(_LANES):
                        d = blk * _LANES + dd
                        vs = [plsc.load_gather(rows_v.at[x, b],
                                               [rvec[m], hb[m] + dd])
                              for m in range(_CHUNK // _LANES)]
                        for m in range(_CHUNK // _LANES):
                            tsp_v[x, b, d, pl.ds(m * _LANES, _LANES)] = vs[m]
                    return carry

                lax.fori_loop(0, D // _LANES, tbody, 0)

        def fire_writes(g, x):
            for b in range(_K):
                s, boff = tile_sb(g * _K + b)
                col = pl.multiple_of(col0 + boff, _CHUNK)
                pltpu.async_copy(tsp_v.at[x, b],
                                 out_hbm.at[s, :, pl.ds(col, _CHUNK)],
                                 osems[x])

        def drain_writes(g, x):
            for b in range(_K):
                s, boff = tile_sb(g * _K + b)
                col = pl.multiple_of(col0 + boff, _CHUNK)
                pltpu.make_async_copy(
                    tsp_v.at[x, b],
                    out_hbm.at[s, :, pl.ds(col, _CHUNK)], osems[x]).wait()

        fire_gathers(0, 0)
        fire_gathers(1, 1)

        def body(g, carry):
            for x in range(2):
                @pl.when(lax.rem(g, 2) == x)
                def _():
                    drain_gathers(g, x)

                    @pl.when(g >= 2)
                    def _():
                        drain_writes(g - 2, x)

                    transpose_group(g, x)
                    fire_writes(g, x)

                    @pl.when(g + 2 < n_groups)
                    def _():
                        fire_gathers(g + 2, x)
            return carry

        lax.fori_loop(0, n_groups, body, 0)

        drain_writes(n_groups - 2, (n_groups - 2) % 2)
        drain_writes(n_groups - 1, (n_groups - 1) % 2)

    return phase_b


def kernel(token_ids, weight):
    B0, S = token_ids.shape
    V, D = weight.shape
    wt = weight.T
    tail = lax.slice(wt, (0, V - _CHUNK), (D, V))
    g = _make_phase_a(V, D)(wt, tail)
    g_rows = g.shape[0]

    tids = token_ids.T.astype(jnp.int32)
    b_per_w = B0 // _NUM_WORKERS
    idx3 = (tids.reshape(S, _NUM_WORKERS, b_per_w // _CHUNK, _CHUNK)
            .transpose(1, 0, 2, 3)
            .reshape(_NUM_WORKERS, S * (b_per_w // _CHUNK), _CHUNK))
    out_t = _make_phase_b(V, D, S, B0, g_rows)(idx3, g)
    return jnp.transpose(out_t, (2, 0, 1))


# speedup vs baseline: 1.5801x; 1.0560x over previous
import functools

import jax
import jax.numpy as jnp
from jax import lax
from jax.experimental import pallas as pl
from jax.experimental.pallas import tpu as pltpu
from jax.experimental.pallas import tpu_sc as plsc

_NUM_CORES = 2
_NUM_SUBCORES = 16
_NUM_WORKERS = _NUM_CORES * _NUM_SUBCORES
_CHUNK = 128
_K = 2
_LANES = 16


@functools.lru_cache(maxsize=None)
def _make_phase_a(V, D):
    n_slabs = -(-V // _CHUNK)
    n_iter = -(-n_slabs // _NUM_WORKERS)
    g_rows = V // 2
    mesh = plsc.VectorSubcoreMesh(core_axis_name="c", subcore_axis_name="s")

    @functools.partial(
        pl.kernel,
        mesh=mesh,
        out_type=jax.ShapeDtypeStruct((g_rows, 2 * D), jnp.float32),
        scratch_types=[
            pltpu.VMEM((2, D, _CHUNK), jnp.float32),
            pltpu.VMEM((2, _CHUNK // 2, 2 * D), jnp.float32),
            pltpu.SemaphoreType.DMA,
            pltpu.SemaphoreType.DMA,
        ],
        compiler_params=pltpu.CompilerParams(
            use_tc_tiling_on_sc=True, needs_layout_passes=False),
    )
    def phase_a(wt_hbm, tail_hbm, g_hbm, slab_v, pack_v, ls, ws):
        wid = lax.axis_index("s") * _NUM_CORES + lax.axis_index("c")

        lane = lax.iota(jnp.int32, _LANES)

        def c_of(t):
            return wid + t * _NUM_WORKERS

        def fire_load(t, buf):
            c = c_of(t)

            @pl.when(c < n_slabs - 1)
            def _():
                col = pl.multiple_of(c * _CHUNK, _CHUNK)
                pltpu.async_copy(wt_hbm.at[:, pl.ds(col, _CHUNK)],
                                 slab_v.at[buf], ls)

            @pl.when(c == n_slabs - 1)
            def _():
                pltpu.async_copy(tail_hbm, slab_v.at[buf], ls)

        def wait_load(t, buf):
            pltpu.make_async_copy(tail_hbm, slab_v.at[buf], ls).wait()

        def row0_of(t):
            c = c_of(t)
            return pl.multiple_of(
                lax.min(c * (_CHUNK // 2), (V - _CHUNK) // 2), 8)

        def fire_store(t, buf):
            pltpu.async_copy(pack_v.at[buf],
                             g_hbm.at[pl.ds(row0_of(t), _CHUNK // 2)], ws)

        def wait_store(t, buf):
            pltpu.make_async_copy(
                pack_v.at[buf],
                g_hbm.at[pl.ds(row0_of(t), _CHUNK // 2)], ws).wait()

        def transpose_slab(buf):
            def tbody(blk, carry):
                dv = lane + blk * _LANES
                off = blk * _LANES
                for j0 in range(0, _CHUNK, 16):
                    vs = []
                    for j in range(j0, j0 + 16):
                        jv = jnp.zeros((_LANES,), jnp.int32) + j
                        vs.append(plsc.load_gather(slab_v.at[buf], [dv, jv]))
                    for i, j in enumerate(range(j0, j0 + 16)):
                        pack_v[buf, j // 2,
                               pl.ds((j % 2) * D + off, _LANES)] = vs[i]
                return carry

            lax.fori_loop(0, D // _LANES, tbody, 0)

        c0_valid = wid < n_slabs

        @pl.when(c0_valid)
        def _():
            fire_load(0, 0)

        def step(t, carry):
            c = c_of(t)
            valid = c < n_slabs
            nxt_valid = c + _NUM_WORKERS < n_slabs

            @pl.when(valid)
            def _():
                for b in range(2):
                    @pl.when(lax.rem(t, 2) == b)
                    def _():
                        wait_load(t, b)

                        @pl.when(nxt_valid)
                        def _():
                            fire_load(t + 1, 1 - b)

                        @pl.when(t >= 2)
                        def _():
                            wait_store(t - 2, b)

                        transpose_slab(b)
                        fire_store(t, b)
            return carry

        lax.fori_loop(0, n_iter, step, 0)

        for t in range(max(0, n_iter - 3), n_iter):
            c = c_of(t)
            c2 = c_of(t + 2)

            @pl.when((c < n_slabs) & (c2 >= n_slabs))
            def _():
                for b in range(2):
                    @pl.when(lax.rem(t, 2) == b)
                    def _():
                        wait_store(t, b)

    return phase_a


@functools.lru_cache(maxsize=None)
def _make_phase_b(V, D, S, B0, g_rows):
    b_per_w = B0 // _NUM_WORKERS
    k_per_s = b_per_w // _CHUNK
    n_tiles = S * k_per_s
    n_groups = n_tiles // _K
    assert n_groups >= 4
    mesh = plsc.VectorSubcoreMesh(core_axis_name="c", subcore_axis_name="s")

    @functools.partial(
        pl.kernel,
        mesh=mesh,
        out_type=jax.ShapeDtypeStruct((S, D, B0), jnp.float32),
        scratch_types=[
            pltpu.VMEM((n_tiles, _CHUNK), jnp.int32),
            pltpu.VMEM((2, _K, _CHUNK), jnp.int32),
            pltpu.VMEM((2, _K, _CHUNK, 2 * D), jnp.float32),
            pltpu.VMEM((2, _K, D, _CHUNK), jnp.float32),
            pltpu.SemaphoreType.DMA,
            pltpu.SemaphoreType.DMA,
            pltpu.SemaphoreType.DMA,
            pltpu.SemaphoreType.DMA,
        ],
        compiler_params=pltpu.CompilerParams(
            use_tc_tiling_on_sc=True, needs_layout_passes=False),
    )
    def phase_b(idx_hbm, g_hbm, out_hbm, idx_v, pidx_v, rows_v, tsp_v,
                gs0, gs1, os0, os1):
        wid = lax.axis_index("s") * _NUM_CORES + lax.axis_index("c")
        col0 = wid * b_per_w
        gsems = (gs0, gs1)
        osems = (os0, os1)
        pltpu.sync_copy(idx_hbm.at[wid], idx_v)

        lane = lax.iota(jnp.int32, _LANES)
        rvec = [lane + m * _LANES for m in range(_CHUNK // _LANES)]

        def tile_sb(t):
            return t // k_per_s, (t % k_per_s) * _CHUNK

        def fire_gathers(g, x):
            for b in range(_K):
                t = g * _K + b
                for m in range(_CHUNK // _LANES):
                    ids = idx_v[t, pl.ds(m * _LANES, _LANES)]
                    pidx_v[x, b, pl.ds(m * _LANES, _LANES)] = (
                        lax.shift_right_logical(ids, 1))
                pltpu.async_copy(g_hbm.at[pidx_v.at[x, b]],
                                 rows_v.at[x, b], gsems[x])

        def drain_gathers(g, x):
            for b in range(_K):
                pltpu.make_async_copy(g_hbm.at[pidx_v.at[x, b]],
                                      rows_v.at[x, b], gsems[x]).wait()

        def transpose_group(g, x):
            for b in range(_K):
                t = g * _K + b
                half = []
                for m in range(_CHUNK // _LANES):
                    ids = idx_v[t, pl.ds(m * _LANES, _LANES)]
                    half.append((ids & 1) * D)

                def tbody(blk, carry):
                    hb = [h + blk * _LANES for h in half]
                    for dd in range(0, _LANES, 2):
                        d = blk * _LANES + dd
                        vs = [plsc.load_gather(rows_v.at[x, b],
                                               [rvec[m], hb[m] + dd + (i // 8)])
                              for i, m in enumerate(
                                  list(range(_CHUNK // _LANES)) * 2)]
                        for i, m in enumerate(
                                list(range(_CHUNK // _LANES)) * 2):
                            tsp_v[x, b, d + i // 8,
                                  pl.ds(m * _LANES, _LANES)] = vs[i]
                    return carry

                lax.fori_loop(0, D // _LANES, tbody, 0)

        def fire_writes(g, x):
            for b in range(_K):
                s, boff = tile_sb(g * _K + b)
                col = pl.multiple_of(col0 + boff, _CHUNK)
                pltpu.async_copy(tsp_v.at[x, b],
                                 out_hbm.at[s, :, pl.ds(col, _CHUNK)],
                                 osems[x])

        def drain_writes(g, x):
            for b in range(_K):
                s, boff = tile_sb(g * _K + b)
                col = pl.multiple_of(col0 + boff, _CHUNK)
                pltpu.make_async_copy(
                    tsp_v.at[x, b],
                    out_hbm.at[s, :, pl.ds(col, _CHUNK)], osems[x]).wait()

        fire_gathers(0, 0)
        fire_gathers(1, 1)

        def body(g, carry):
            for x in range(2):
                @pl.when(lax.rem(g, 2) == x)
                def _():
                    drain_gathers(g, x)

                    @pl.when(g >= 2)
                    def _():
                        drain_writes(g - 2, x)

                    transpose_group(g, x)
                    fire_writes(g, x)

                    @pl.when(g + 2 < n_groups)
                    def _():
                        fire_gathers(g + 2, x)
            return carry

        lax.fori_loop(0, n_groups, body, 0)

        drain_writes(n_groups - 2, (n_groups - 2) % 2)
        drain_writes(n_groups - 1, (n_groups - 1) % 2)

    return phase_b


def kernel(token_ids, weight):
    B0, S = token_ids.shape
    V, D = weight.shape
    wt = weight.T
    tail = lax.slice(wt, (0, V - _CHUNK), (D, V))
    g = _make_phase_a(V, D)(wt, tail)
    g_rows = g.shape[0]

    tids = token_ids.T.astype(jnp.int32)
    b_per_w = B0 // _NUM_WORKERS
    idx3 = (tids.reshape(S, _NUM_WORKERS, b_per_w // _CHUNK, _CHUNK)
            .transpose(1, 0, 2, 3)
            .reshape(_NUM_WORKERS, S * (b_per_w // _CHUNK), _CHUNK))
    out_t = _make_phase_b(V, D, S, B0, g_rows)(idx3, g)
    return jnp.transpose(out_t, (2, 0, 1))


# speedup vs baseline: 2.2662x; 1.4342x over previous
import functools

import jax
import jax.numpy as jnp
from jax import lax
from jax.experimental import pallas as pl
from jax.experimental.pallas import tpu as pltpu
from jax.experimental.pallas import tpu_sc as plsc

_NUM_CORES = 2
_NUM_SUBCORES = 16
_NUM_WORKERS = _NUM_CORES * _NUM_SUBCORES
_CHUNK = 128


_K = 4


@functools.lru_cache(maxsize=None)
def _make_gather(V, D, B):
    n_per_w = B // _NUM_WORKERS
    n_chunks = n_per_w // _CHUNK
    n_groups = n_chunks // _K
    assert n_groups % 2 == 0 and n_groups >= 4
    mesh = plsc.VectorSubcoreMesh(core_axis_name="c", subcore_axis_name="s")

    @functools.partial(
        pl.kernel,
        mesh=mesh,
        out_type=jax.ShapeDtypeStruct((B, D), jnp.float32),
        scratch_types=[
            pltpu.VMEM((n_chunks, _CHUNK), jnp.int32),
            pltpu.VMEM((_K, _CHUNK, D), jnp.float32),
            pltpu.VMEM((_K, _CHUNK, D), jnp.float32),
            pltpu.SemaphoreType.DMA,
            pltpu.SemaphoreType.DMA,
            pltpu.SemaphoreType.DMA,
            pltpu.SemaphoreType.DMA,
        ],
        compiler_params=pltpu.CompilerParams(use_tc_tiling_on_sc=False),
    )
    def gather_kernel(idx_hbm, table_hbm, out_hbm, idx_v,
                      rows_a, rows_b, gs_a, gs_b, os_a, os_b):
        wid = lax.axis_index("s") * _NUM_CORES + lax.axis_index("c")
        out_base = wid * n_per_w
        pltpu.sync_copy(idx_hbm.at[wid], idx_v)

        def fire_gathers(g, rows, sem):
            for b in range(_K):
                pltpu.async_copy(table_hbm.at[idx_v.at[g * _K + b]],
                                 rows.at[b], sem)

        def drain_gathers(g, rows, sem):
            for b in range(_K):
                pltpu.make_async_copy(table_hbm.at[idx_v.at[g * _K + b]],
                                      rows.at[b], sem).wait()

        def fire_scatters(g, rows, sem):
            for b in range(_K):
                dst = out_hbm.at[pl.ds(out_base + (g * _K + b) * _CHUNK, _CHUNK)]
                pltpu.async_copy(rows.at[b], dst, sem)

        def drain_scatters(g, rows, sem):
            for b in range(_K):
                dst = out_hbm.at[pl.ds(out_base + (g * _K + b) * _CHUNK, _CHUNK)]
                pltpu.make_async_copy(rows.at[b], dst, sem).wait()

        fire_gathers(0, rows_a, gs_a)

        def body(t, carry):
            ga, gb = 2 * t, 2 * t + 1
            fire_gathers(gb, rows_b, gs_b)
            drain_gathers(ga, rows_a, gs_a)
            fire_scatters(ga, rows_a, os_a)
            drain_gathers(gb, rows_b, gs_b)
            fire_scatters(gb, rows_b, os_b)
            drain_scatters(ga, rows_a, os_a)
            fire_gathers(ga + 2, rows_a, gs_a)
            drain_scatters(gb, rows_b, os_b)
            return carry

        lax.fori_loop(0, n_groups // 2 - 1, body, 0)

        g_last = n_groups - 2
        fire_gathers(g_last + 1, rows_b, gs_b)
        drain_gathers(g_last, rows_a, gs_a)
        fire_scatters(g_last, rows_a, os_a)
        drain_gathers(g_last + 1, rows_b, gs_b)
        fire_scatters(g_last + 1, rows_b, os_b)
        drain_scatters(g_last, rows_a, os_a)
        drain_scatters(g_last + 1, rows_b, os_b)

    return gather_kernel


def kernel(token_ids, weight):
    B0, S = token_ids.shape
    V, D = weight.shape
    B = B0 * S
    idx = token_ids.reshape(
        _NUM_WORKERS, B // (_NUM_WORKERS * _CHUNK), _CHUNK
    ).astype(jnp.int32)
    out = _make_gather(V, D, B)(idx, weight)
    return out.reshape(B0, S, D)
